# Initial kernel scaffold; baseline (speedup 1.0000x reference)
import jax
import jax.numpy as jnp
from jax.experimental import pallas as pl


def kernel(h_lig, h_kp, x_lig, x_kp, z_lig, edge_index_ll, edge_index_kl, params):
    raise NotImplementedError("write your pallas kernel here")


# speedup vs baseline: 3.3272x; 3.3272x over previous
import dataclasses
import functools

import jax
import jax.numpy as jnp
from jax import lax
from jax.experimental import pallas as pl
from jax.experimental.pallas import tpu as pltpu
from jax.experimental.pallas import tpu_sc as plsc

NC, NS = 2, 16
NW = NC * NS
SCB = 80
XW = 16
L = 16
NCHUNK = SCB // L


def _iota():
    return lax.iota(jnp.int32, L)


def _sc_params():
    cp = pltpu.CompilerParams()
    if "needs_layout_passes" in pltpu.CompilerParams.__dataclass_fields__:
        cp = dataclasses.replace(cp, needs_layout_passes=False)
    return cp


def _sc_gather(hs_t, hd_t, xs4_t, xd4_t, s_idx, d_idx):
    E = s_idx.shape[0]
    D = hs_t.shape[1]
    ew = E // NW
    nblk = ew // SCB
    mesh = plsc.VectorSubcoreMesh(core_axis_name="c", subcore_axis_name="s")
    f32 = jnp.float32

    @functools.partial(
        pl.kernel,
        out_type=(
            jax.ShapeDtypeStruct((E, D), f32),
            jax.ShapeDtypeStruct((E, D), f32),
            jax.ShapeDtypeStruct((E * XW,), f32),
            jax.ShapeDtypeStruct((E * XW,), f32),
        ),
        mesh=mesh,
        scratch_types=[
            pltpu.VMEM((SCB,), jnp.int32),
            pltpu.VMEM((SCB,), jnp.int32),
            pltpu.VMEM((SCB, D), f32),
            pltpu.VMEM((SCB, D), f32),
            pltpu.VMEM((SCB * XW,), f32),
            pltpu.VMEM((SCB * XW,), f32),
            pltpu.VMEM((xs4_t.shape[0],), f32),
            pltpu.VMEM((xd4_t.shape[0],), f32),
            pltpu.SemaphoreType.DMA,
        ],
        compiler_params=_sc_params(),
    )
    def k(hs_hbm, hd_hbm, xs4_hbm, xd4_hbm, si_hbm, di_hbm,
          hs_o, hd_o, xsf_o, xdf_o,
          si_v, di_v, hs_b, hd_b, xs_b, xd_b, xs_tile, xd_tile, sem):
        wid = lax.axis_index("s") * NC + lax.axis_index("c")
        base0 = wid * ew
        pltpu.sync_copy(xs4_hbm, xs_tile)
        pltpu.sync_copy(xd4_hbm, xd_tile)

        @pl.loop(0, SCB * XW // L)
        def _(i):
            xs_b[pl.ds(i * L, L)] = jnp.zeros((L,), f32)
            xd_b[pl.ds(i * L, L)] = jnp.zeros((L,), f32)

        iot = _iota()

        @pl.loop(0, nblk)
        def _(j):
            base = base0 + j * SCB
            pltpu.sync_copy(si_hbm.at[pl.ds(base, SCB)], si_v)
            pltpu.sync_copy(di_hbm.at[pl.ds(base, SCB)], di_v)
            c1 = pltpu.async_copy(hs_hbm.at[si_v], hs_b, sem)
            c2 = pltpu.async_copy(hd_hbm.at[di_v], hd_b, sem)
            for kk in range(NCHUNK):
                sc = si_v[pl.ds(kk * L, L)] << 2
                dc = di_v[pl.ds(kk * L, L)] << 2
                pos = (iot << 4) + (kk * L * XW)
                for c in range(3):
                    vs = plsc.load_gather(xs_tile, [sc + c])
                    plsc.store_scatter(xs_b, [pos + c], vs)
                    vd = plsc.load_gather(xd_tile, [dc + c])
                    plsc.store_scatter(xd_b, [pos + c], vd)
            c1.wait()
            c2.wait()
            pltpu.sync_copy(hs_b, hs_o.at[pl.ds(base, SCB)])
            pltpu.sync_copy(hd_b, hd_o.at[pl.ds(base, SCB)])
            pltpu.sync_copy(xs_b, xsf_o.at[pl.ds(base * XW, SCB * XW)])
            pltpu.sync_copy(xd_b, xdf_o.at[pl.ds(base * XW, SCB * XW)])

    return k(hs_t, hd_t, xs4_t, xd4_t, s_idx, d_idx)


def _edge_tc(hs, hd, xs, xd, w, bt):
    E, D = hs.shape
    f32 = jnp.float32

    def body(hs_r, hd_r, xs_r, xd_r,
             w1a, w1b, w1c, b1, w2, b2, wa, ba,
             wc1a, wc1b, wc1c, bc1, wc2, bc2, wc3,
             mh_o, mx_o):
        diff = xs_r[...] - xd_r[...]
        dij = jnp.sqrt(jnp.sum(diff * diff, axis=1, keepdims=True))
        hsv, hdv = hs_r[...], hd_r[...]
        dot = functools.partial(jnp.dot, preferred_element_type=f32)
        u = dot(hsv, w1a[...]) + dot(hdv, w1b[...]) + dij * w1c[...] + b1[...]
        m = jax.nn.silu(u)
        m2 = jax.nn.silu(dot(m, w2[...]) + b2[...])
        g = jax.nn.sigmoid(jnp.sum(m2 * wa[...], axis=1, keepdims=True)
                           + ba[0, 0])
        mh_o[...] = m2 * g
        v = dot(hsv, wc1a[...]) + dot(hdv, wc1b[...]) + dij * wc1c[...] + bc1[...]
        c = jax.nn.silu(v)
        c2 = jax.nn.silu(dot(c, wc2[...]) + bc2[...])
        cc = jnp.sum(c2 * wc3[...], axis=1, keepdims=True)
        mx_o[...] = cc * diff / (dij + 1.0)

    row = lambda i: (i, 0)
    full = lambda i: (0, 0)
    eb = lambda width: pl.BlockSpec((bt, width), row)
    wspec = lambda a: pl.BlockSpec(a.shape, full)

    w1 = w['W1']
    wc1 = w['Wc1']
    args = (w1[:D], w1[D:2 * D], w1[2 * D:2 * D + 1], w['b1'].reshape(1, D),
            w['W2'], w['b2'].reshape(1, D),
            w['Wa'].reshape(1, D), w['ba'].reshape(1, 1),
            wc1[:D], wc1[D:2 * D], wc1[2 * D:2 * D + 1], w['bc1'].reshape(1, D),
            w['Wc2'], w['bc2'].reshape(1, D), w['Wc3'].reshape(1, D))

    return pl.pallas_call(
        body,
        grid=(E // bt,),
        in_specs=[eb(D), eb(D), eb(XW), eb(XW)] + [wspec(a) for a in args],
        out_specs=[eb(D), eb(XW)],
        out_shape=(jax.ShapeDtypeStruct((E, D), f32),
                   jax.ShapeDtypeStruct((E, XW), f32)),
    )(hs, hd, xs, xd, *args)


def _sc_scatter(mh_ll, mxf_ll, d_ll, mh_kl, mxf_kl, d_kl, n_lig):
    D = mh_ll.shape[1]
    f32 = jnp.float32
    mesh = plsc.VectorSubcoreMesh(core_axis_name="c", subcore_axis_name="s")
    nx = n_lig // 8
    nh_p = -(-n_lig // 128) * 128
    nx_p = -(-nx // 128) * 128
    hr = -(-nh_p // NS) // 8 * 8
    h_parts = [(s * hr, min(hr, nh_p - s * hr)) for s in range(NS)
               if s * hr < nh_p]
    xr = max(8, -(-nx_p // NS) // 8 * 8)
    x_parts = [(s * xr, min(xr, nx_p - s * xr)) for s in range(NS)
               if s * xr < nx_p]
    zeros_h = jnp.zeros((nh_p, D), f32)
    zeros_x = jnp.zeros((nx_p, D), f32)

    @functools.partial(
        pl.kernel,
        out_type=(jax.ShapeDtypeStruct((NC, nh_p, D), f32),
                  jax.ShapeDtypeStruct((NC, nx_p, D), f32)),
        mesh=mesh,
        scratch_types=[
            pltpu.VMEM((SCB,), jnp.int32),
            pltpu.VMEM((SCB,), jnp.int32),
            pltpu.VMEM((SCB, D), f32),
            pltpu.VMEM((SCB * XW,), f32),
            pltpu.VMEM((SCB, D), f32),
            pltpu.VMEM_SHARED((nh_p, D), f32),
            pltpu.VMEM_SHARED((nx_p, D), f32),
        ],
        compiler_params=_sc_params(),
    )
    def k(mhll_hbm, mxll_hbm, dll_hbm, mhkl_hbm, mxkl_hbm, dkl_hbm,
          zh_hbm, zx_hbm, hacc_o, xacc_o,
          di_v, dr_v, mh_v, mx_v, xexp, acc_h, acc_x):
        cid = lax.axis_index("c")
        sid = lax.axis_index("s")
        for s, (off, cnt) in enumerate(h_parts):
            @pl.when(sid == s)
            def _(off=off, cnt=cnt):
                pltpu.sync_copy(zh_hbm.at[pl.ds(off, cnt)],
                                acc_h.at[pl.ds(off, cnt)])
        for s, (off, cnt) in enumerate(x_parts):
            @pl.when(sid == s)
            def _(off=off, cnt=cnt):
                pltpu.sync_copy(zx_hbm.at[pl.ds(off, cnt)],
                                acc_x.at[pl.ds(off, cnt)])

        @pl.loop(0, SCB)
        def _(i):
            for kk in range(D // L):
                xexp[i, pl.ds(kk * L, L)] = jnp.zeros((L,), f32)

        plsc.subcore_barrier()
        iot = _iota()

        for mh_hbm, mxf_hbm, dd_hbm in ((mhll_hbm, mxll_hbm, dll_hbm),
                                        (mhkl_hbm, mxkl_hbm, dkl_hbm)):
            e = dd_hbm.shape[0]
            ew = e // NW
            nblk = ew // SCB
            base0 = cid * (e // NC) + sid * ew

            @pl.loop(0, nblk)
            def _(j):
                base = base0 + j * SCB
                pltpu.sync_copy(dd_hbm.at[pl.ds(base, SCB)], di_v)
                pltpu.sync_copy(mh_hbm.at[pl.ds(base, SCB)], mh_v)
                pltpu.sync_copy(mxf_hbm.at[pl.ds(base * XW, SCB * XW)], mx_v)
                for kk in range(NCHUNK):
                    dc = di_v[pl.ds(kk * L, L)]
                    dr_v[pl.ds(kk * L, L)] = dc >> 3
                    rowi = iot + kk * L
                    slot = (dc & 7) << 4
                    for c in range(3):
                        val = plsc.load_gather(
                            mx_v, [(iot << 4) + (kk * L * XW + c)])
                        plsc.store_scatter(xexp, [rowi, slot + c], val)
                pltpu.sync_copy(mh_v, acc_h.at[di_v], add=True)
                pltpu.sync_copy(xexp, acc_x.at[dr_v], add=True)
                for kk in range(NCHUNK):
                    dc = di_v[pl.ds(kk * L, L)]
                    rowi = iot + kk * L
                    slot = (dc & 7) << 4
                    zv = jnp.zeros((L,), f32)
                    for c in range(3):
                        plsc.store_scatter(xexp, [rowi, slot + c], zv)

        plsc.subcore_barrier()
        for s, (off, cnt) in enumerate(h_parts):
            @pl.when(sid == s)
            def _(off=off, cnt=cnt):
                pltpu.sync_copy(acc_h.at[pl.ds(off, cnt)],
                                hacc_o.at[cid, pl.ds(off, cnt)])
        for s, (off, cnt) in enumerate(x_parts):
            @pl.when(sid == s)
            def _(off=off, cnt=cnt):
                pltpu.sync_copy(acc_x.at[pl.ds(off, cnt)],
                                xacc_o.at[cid, pl.ds(off, cnt)])

    return k(mh_ll, mxf_ll, d_ll, mh_kl, mxf_kl, d_kl, zeros_h, zeros_x)


def _node_tc(h_lig, hacc0, hacc1, xacc0, xacc1, x_pad, z, pn, br):
    n, D = h_lig.shape
    f32 = jnp.float32

    def body(h_r, ha0, ha1, xa0, xa1, xp_r, z_r, wn1a, wn1b, bn1, wn2, bn2,
             nh_o, nx_o):
        zinv = 1.0 / z_r[...]
        hn = (ha0[...] + ha1[...]) * zinv
        xn = (xa0[...] + xa1[...]) * zinv
        hv = h_r[...]
        dot = functools.partial(jnp.dot, preferred_element_type=f32)
        t = jax.nn.silu(dot(hv, wn1a[...]) + dot(hn, wn1b[...]) + bn1[...])
        nh_o[...] = hv + dot(t, wn2[...]) + bn2[...]
        nx_o[...] = xp_r[...] + xn

    row = lambda i: (i, 0)
    full = lambda i: (0, 0)
    rb = lambda width: pl.BlockSpec((br, width), row)
    wspec = lambda a: pl.BlockSpec(a.shape, full)

    wn1 = pn['Wn1']
    args = (wn1[:D], wn1[D:], pn['bn1'].reshape(1, D), pn['Wn2'],
            pn['bn2'].reshape(1, D))

    return pl.pallas_call(
        body,
        grid=(n // br,),
        in_specs=[rb(D), rb(D), rb(D), rb(XW), rb(XW), rb(XW),
                  pl.BlockSpec((br, 1), row)] + [wspec(a) for a in args],
        out_specs=[rb(D), rb(XW)],
        out_shape=(jax.ShapeDtypeStruct((n, D), f32),
                   jax.ShapeDtypeStruct((n, XW), f32)),
    )(h_lig, hacc0, hacc1, xacc0, xacc1, x_pad, z, *args)


def kernel(h_lig, h_kp, x_lig, x_kp, z_lig, edge_index_ll, edge_index_kl,
           params):
    n_lig, D = h_lig.shape
    f32 = jnp.float32
    xdim = x_lig.shape[1]

    xl_pad = jnp.pad(x_lig.astype(f32), ((0, 0), (0, XW - xdim)))
    xk_pad = jnp.pad(x_kp.astype(f32), ((0, 0), (0, XW - xdim)))
    xl4 = jnp.pad(x_lig.astype(f32), ((0, 0), (0, 4 - xdim))).reshape(-1)
    xk4 = jnp.pad(x_kp.astype(f32), ((0, 0), (0, 4 - xdim))).reshape(-1)

    s_ll = edge_index_ll[0].astype(jnp.int32)
    d_ll = edge_index_ll[1].astype(jnp.int32)
    s_kl = edge_index_kl[0].astype(jnp.int32)
    d_kl = edge_index_kl[1].astype(jnp.int32)
    e_ll = s_ll.shape[0]
    e_kl = s_kl.shape[0]

    hs_ll, hd_ll, xsf_ll, xdf_ll = _sc_gather(h_lig, h_lig, xl4, xl4,
                                              s_ll, d_ll)
    hs_kl, hd_kl, xsf_kl, xdf_kl = _sc_gather(h_kp, h_lig, xk4, xl4,
                                              s_kl, d_kl)

    mh_ll, mx_ll = _edge_tc(hs_ll, hd_ll, xsf_ll.reshape(e_ll, XW),
                            xdf_ll.reshape(e_ll, XW), params['ll'], 1600)
    mh_kl, mx_kl = _edge_tc(hs_kl, hd_kl, xsf_kl.reshape(e_kl, XW),
                            xdf_kl.reshape(e_kl, XW), params['kl'], 1600)

    hacc, xacc = _sc_scatter(mh_ll, mx_ll.reshape(-1), d_ll,
                             mh_kl, mx_kl.reshape(-1), d_kl, n_lig)

    xacc0 = xacc[0, :n_lig // 8].reshape(n_lig, XW)
    xacc1 = xacc[1, :n_lig // 8].reshape(n_lig, XW)
    new_h, new_x_pad = _node_tc(h_lig, hacc[0, :n_lig], hacc[1, :n_lig],
                                xacc0, xacc1,
                                xl_pad, z_lig, params['node'], 2000)
    return new_h, new_x_pad[:, :xdim]


# speedup vs baseline: 3.4895x; 1.0488x over previous
import dataclasses
import functools

import jax
import jax.numpy as jnp
from jax import lax
from jax.experimental import pallas as pl
from jax.experimental.pallas import tpu as pltpu
from jax.experimental.pallas import tpu_sc as plsc

NC, NS = 2, 16
NW = NC * NS
SCB = 80
XW = 16
L = 16
NCHUNK = SCB // L


def _iota():
    return lax.iota(jnp.int32, L)


def _sc_params():
    cp = pltpu.CompilerParams()
    if "needs_layout_passes" in pltpu.CompilerParams.__dataclass_fields__:
        cp = dataclasses.replace(cp, needs_layout_passes=False)
    return cp


def _sc_gather(hs_t, hd_t, xs4_t, xd4_t, s_idx, d_idx):
    E = s_idx.shape[0]
    D = hs_t.shape[1]
    ew = E // NW
    nblk = ew // SCB
    mesh = plsc.VectorSubcoreMesh(core_axis_name="c", subcore_axis_name="s")
    f32 = jnp.float32

    @functools.partial(
        pl.kernel,
        out_type=(
            jax.ShapeDtypeStruct((E, D), f32),
            jax.ShapeDtypeStruct((E, D), f32),
            jax.ShapeDtypeStruct((E * XW,), f32),
            jax.ShapeDtypeStruct((E * XW,), f32),
        ),
        mesh=mesh,
        scratch_types=[
            pltpu.VMEM((SCB,), jnp.int32),
            pltpu.VMEM((SCB,), jnp.int32),
            pltpu.VMEM((SCB, D), f32),
            pltpu.VMEM((SCB, D), f32),
            pltpu.VMEM((SCB * XW,), f32),
            pltpu.VMEM((SCB * XW,), f32),
            pltpu.VMEM((xs4_t.shape[0],), f32),
            pltpu.VMEM((xd4_t.shape[0],), f32),
            pltpu.SemaphoreType.DMA,
        ],
        compiler_params=_sc_params(),
    )
    def k(hs_hbm, hd_hbm, xs4_hbm, xd4_hbm, si_hbm, di_hbm,
          hs_o, hd_o, xsf_o, xdf_o,
          si_v, di_v, hs_b, hd_b, xs_b, xd_b, xs_tile, xd_tile, sem):
        wid = lax.axis_index("s") * NC + lax.axis_index("c")
        base0 = wid * ew
        pltpu.sync_copy(xs4_hbm, xs_tile)
        pltpu.sync_copy(xd4_hbm, xd_tile)

        @pl.loop(0, SCB * XW // L)
        def _(i):
            xs_b[pl.ds(i * L, L)] = jnp.zeros((L,), f32)
            xd_b[pl.ds(i * L, L)] = jnp.zeros((L,), f32)

        iot = _iota()

        @pl.loop(0, nblk)
        def _(j):
            base = base0 + j * SCB
            pltpu.sync_copy(si_hbm.at[pl.ds(base, SCB)], si_v)
            pltpu.sync_copy(di_hbm.at[pl.ds(base, SCB)], di_v)
            c1 = pltpu.async_copy(hs_hbm.at[si_v], hs_b, sem)
            c2 = pltpu.async_copy(hd_hbm.at[di_v], hd_b, sem)
            for kk in range(NCHUNK):
                sc = si_v[pl.ds(kk * L, L)] << 2
                dc = di_v[pl.ds(kk * L, L)] << 2
                pos = (iot << 4) + (kk * L * XW)
                for c in range(3):
                    vs = plsc.load_gather(xs_tile, [sc + c])
                    plsc.store_scatter(xs_b, [pos + c], vs)
                    vd = plsc.load_gather(xd_tile, [dc + c])
                    plsc.store_scatter(xd_b, [pos + c], vd)
            c1.wait()
            c2.wait()
            pltpu.sync_copy(hs_b, hs_o.at[pl.ds(base, SCB)])
            pltpu.sync_copy(hd_b, hd_o.at[pl.ds(base, SCB)])
            pltpu.sync_copy(xs_b, xsf_o.at[pl.ds(base * XW, SCB * XW)])
            pltpu.sync_copy(xd_b, xdf_o.at[pl.ds(base * XW, SCB * XW)])

    return k(hs_t, hd_t, xs4_t, xd4_t, s_idx, d_idx)


def _edge_tc(hs, hd, xs, xd, w, bt):
    E, D = hs.shape
    f32 = jnp.float32

    def body(hs_r, hd_r, xs_r, xd_r,
             w1a, w1b, w1c, b1, w2, b2, wa, ba,
             wc1a, wc1b, wc1c, bc1, wc2, bc2, wc3,
             mh_o, mx_o):
        diff = xs_r[...] - xd_r[...]
        dij = jnp.sqrt(jnp.sum(diff * diff, axis=1, keepdims=True))
        hsv, hdv = hs_r[...], hd_r[...]
        bf = jnp.bfloat16
        dot = lambda a, b: jnp.dot(a.astype(bf), b.astype(bf),
                                   preferred_element_type=f32)
        u = dot(hsv, w1a[...]) + dot(hdv, w1b[...]) + dij * w1c[...] + b1[...]
        m = jax.nn.silu(u)
        m2 = jax.nn.silu(dot(m, w2[...]) + b2[...])
        g = jax.nn.sigmoid(jnp.sum(m2 * wa[...], axis=1, keepdims=True)
                           + ba[0, 0])
        mh_o[...] = m2 * g
        v = dot(hsv, wc1a[...]) + dot(hdv, wc1b[...]) + dij * wc1c[...] + bc1[...]
        c = jax.nn.silu(v)
        c2 = jax.nn.silu(dot(c, wc2[...]) + bc2[...])
        cc = jnp.sum(c2 * wc3[...], axis=1, keepdims=True)
        mx_o[...] = cc * diff / (dij + 1.0)

    row = lambda i: (i, 0)
    full = lambda i: (0, 0)
    eb = lambda width: pl.BlockSpec((bt, width), row)
    wspec = lambda a: pl.BlockSpec(a.shape, full)

    w1 = w['W1']
    wc1 = w['Wc1']
    args = (w1[:D], w1[D:2 * D], w1[2 * D:2 * D + 1], w['b1'].reshape(1, D),
            w['W2'], w['b2'].reshape(1, D),
            w['Wa'].reshape(1, D), w['ba'].reshape(1, 1),
            wc1[:D], wc1[D:2 * D], wc1[2 * D:2 * D + 1], w['bc1'].reshape(1, D),
            w['Wc2'], w['bc2'].reshape(1, D), w['Wc3'].reshape(1, D))

    return pl.pallas_call(
        body,
        grid=(E // bt,),
        in_specs=[eb(D), eb(D), eb(XW), eb(XW)] + [wspec(a) for a in args],
        out_specs=[eb(D), eb(XW)],
        out_shape=(jax.ShapeDtypeStruct((E, D), f32),
                   jax.ShapeDtypeStruct((E, XW), f32)),
    )(hs, hd, xs, xd, *args)


def _sc_scatter(mh_ll, mxf_ll, d_ll, mh_kl, mxf_kl, d_kl, n_lig):
    D = mh_ll.shape[1]
    f32 = jnp.float32
    mesh = plsc.VectorSubcoreMesh(core_axis_name="c", subcore_axis_name="s")
    nx = n_lig // 8
    nh_p = -(-n_lig // 128) * 128
    nx_p = -(-nx // 128) * 128
    hr = -(-nh_p // NS) // 8 * 8
    h_parts = [(s * hr, min(hr, nh_p - s * hr)) for s in range(NS)
               if s * hr < nh_p]
    xr = max(8, -(-nx_p // NS) // 8 * 8)
    x_parts = [(s * xr, min(xr, nx_p - s * xr)) for s in range(NS)
               if s * xr < nx_p]
    zeros_h = jnp.zeros((nh_p, D), f32)
    zeros_x = jnp.zeros((nx_p, D), f32)

    @functools.partial(
        pl.kernel,
        out_type=(jax.ShapeDtypeStruct((NC, nh_p, D), f32),
                  jax.ShapeDtypeStruct((NC, nx_p, D), f32)),
        mesh=mesh,
        scratch_types=[
            pltpu.VMEM((SCB,), jnp.int32),
            pltpu.VMEM((SCB,), jnp.int32),
            pltpu.VMEM((SCB, D), f32),
            pltpu.VMEM((SCB * XW,), f32),
            pltpu.VMEM((SCB, D), f32),
            pltpu.VMEM_SHARED((nh_p, D), f32),
            pltpu.VMEM_SHARED((nx_p, D), f32),
        ],
        compiler_params=_sc_params(),
    )
    def k(mhll_hbm, mxll_hbm, dll_hbm, mhkl_hbm, mxkl_hbm, dkl_hbm,
          zh_hbm, zx_hbm, hacc_o, xacc_o,
          di_v, dr_v, mh_v, mx_v, xexp, acc_h, acc_x):
        cid = lax.axis_index("c")
        sid = lax.axis_index("s")
        for s, (off, cnt) in enumerate(h_parts):
            @pl.when(sid == s)
            def _(off=off, cnt=cnt):
                pltpu.sync_copy(zh_hbm.at[pl.ds(off, cnt)],
                                acc_h.at[pl.ds(off, cnt)])
        for s, (off, cnt) in enumerate(x_parts):
            @pl.when(sid == s)
            def _(off=off, cnt=cnt):
                pltpu.sync_copy(zx_hbm.at[pl.ds(off, cnt)],
                                acc_x.at[pl.ds(off, cnt)])

        @pl.loop(0, SCB)
        def _(i):
            for kk in range(D // L):
                xexp[i, pl.ds(kk * L, L)] = jnp.zeros((L,), f32)

        plsc.subcore_barrier()
        iot = _iota()

        for mh_hbm, mxf_hbm, dd_hbm in ((mhll_hbm, mxll_hbm, dll_hbm),
                                        (mhkl_hbm, mxkl_hbm, dkl_hbm)):
            e = dd_hbm.shape[0]
            ew = e // NW
            nblk = ew // SCB
            base0 = cid * (e // NC) + sid * ew

            @pl.loop(0, nblk)
            def _(j):
                base = base0 + j * SCB
                pltpu.sync_copy(dd_hbm.at[pl.ds(base, SCB)], di_v)
                pltpu.sync_copy(mh_hbm.at[pl.ds(base, SCB)], mh_v)
                pltpu.sync_copy(mxf_hbm.at[pl.ds(base * XW, SCB * XW)], mx_v)
                for kk in range(NCHUNK):
                    dc = di_v[pl.ds(kk * L, L)]
                    dr_v[pl.ds(kk * L, L)] = dc >> 3
                    rowi = iot + kk * L
                    slot = (dc & 7) << 4
                    for c in range(3):
                        val = plsc.load_gather(
                            mx_v, [(iot << 4) + (kk * L * XW + c)])
                        plsc.store_scatter(xexp, [rowi, slot + c], val)
                pltpu.sync_copy(mh_v, acc_h.at[di_v], add=True)
                pltpu.sync_copy(xexp, acc_x.at[dr_v], add=True)
                for kk in range(NCHUNK):
                    dc = di_v[pl.ds(kk * L, L)]
                    rowi = iot + kk * L
                    slot = (dc & 7) << 4
                    zv = jnp.zeros((L,), f32)
                    for c in range(3):
                        plsc.store_scatter(xexp, [rowi, slot + c], zv)

        plsc.subcore_barrier()
        for s, (off, cnt) in enumerate(h_parts):
            @pl.when(sid == s)
            def _(off=off, cnt=cnt):
                pltpu.sync_copy(acc_h.at[pl.ds(off, cnt)],
                                hacc_o.at[cid, pl.ds(off, cnt)])
        for s, (off, cnt) in enumerate(x_parts):
            @pl.when(sid == s)
            def _(off=off, cnt=cnt):
                pltpu.sync_copy(acc_x.at[pl.ds(off, cnt)],
                                xacc_o.at[cid, pl.ds(off, cnt)])

    return k(mh_ll, mxf_ll, d_ll, mh_kl, mxf_kl, d_kl, zeros_h, zeros_x)


def _node_tc(h_lig, hacc0, hacc1, xacc0, xacc1, x_pad, z, pn, br):
    n, D = h_lig.shape
    f32 = jnp.float32

    def body(h_r, ha0, ha1, xa0, xa1, xp_r, z_r, wn1a, wn1b, bn1, wn2, bn2,
             nh_o, nx_o):
        zinv = 1.0 / z_r[...]
        hn = (ha0[...] + ha1[...]) * zinv
        xn = (xa0[...] + xa1[...]) * zinv
        hv = h_r[...]
        dot = functools.partial(jnp.dot, preferred_element_type=f32)
        t = jax.nn.silu(dot(hv, wn1a[...]) + dot(hn, wn1b[...]) + bn1[...])
        nh_o[...] = hv + dot(t, wn2[...]) + bn2[...]
        nx_o[...] = xp_r[...] + xn

    row = lambda i: (i, 0)
    full = lambda i: (0, 0)
    rb = lambda width: pl.BlockSpec((br, width), row)
    wspec = lambda a: pl.BlockSpec(a.shape, full)

    wn1 = pn['Wn1']
    args = (wn1[:D], wn1[D:], pn['bn1'].reshape(1, D), pn['Wn2'],
            pn['bn2'].reshape(1, D))

    return pl.pallas_call(
        body,
        grid=(n // br,),
        in_specs=[rb(D), rb(D), rb(D), rb(XW), rb(XW), rb(XW),
                  pl.BlockSpec((br, 1), row)] + [wspec(a) for a in args],
        out_specs=[rb(D), rb(XW)],
        out_shape=(jax.ShapeDtypeStruct((n, D), f32),
                   jax.ShapeDtypeStruct((n, XW), f32)),
    )(h_lig, hacc0, hacc1, xacc0, xacc1, x_pad, z, *args)


def kernel(h_lig, h_kp, x_lig, x_kp, z_lig, edge_index_ll, edge_index_kl,
           params):
    n_lig, D = h_lig.shape
    f32 = jnp.float32
    xdim = x_lig.shape[1]

    xl_pad = jnp.pad(x_lig.astype(f32), ((0, 0), (0, XW - xdim)))
    xk_pad = jnp.pad(x_kp.astype(f32), ((0, 0), (0, XW - xdim)))
    xl4 = jnp.pad(x_lig.astype(f32), ((0, 0), (0, 4 - xdim))).reshape(-1)
    xk4 = jnp.pad(x_kp.astype(f32), ((0, 0), (0, 4 - xdim))).reshape(-1)

    s_ll = edge_index_ll[0].astype(jnp.int32)
    d_ll = edge_index_ll[1].astype(jnp.int32)
    s_kl = edge_index_kl[0].astype(jnp.int32)
    d_kl = edge_index_kl[1].astype(jnp.int32)
    e_ll = s_ll.shape[0]
    e_kl = s_kl.shape[0]

    hs_ll, hd_ll, xsf_ll, xdf_ll = _sc_gather(h_lig, h_lig, xl4, xl4,
                                              s_ll, d_ll)
    hs_kl, hd_kl, xsf_kl, xdf_kl = _sc_gather(h_kp, h_lig, xk4, xl4,
                                              s_kl, d_kl)

    mh_ll, mx_ll = _edge_tc(hs_ll, hd_ll, xsf_ll.reshape(e_ll, XW),
                            xdf_ll.reshape(e_ll, XW), params['ll'], 6400)
    mh_kl, mx_kl = _edge_tc(hs_kl, hd_kl, xsf_kl.reshape(e_kl, XW),
                            xdf_kl.reshape(e_kl, XW), params['kl'], 6400)

    hacc, xacc = _sc_scatter(mh_ll, mx_ll.reshape(-1), d_ll,
                             mh_kl, mx_kl.reshape(-1), d_kl, n_lig)

    xacc0 = xacc[0, :n_lig // 8].reshape(n_lig, XW)
    xacc1 = xacc[1, :n_lig // 8].reshape(n_lig, XW)
    new_h, new_x_pad = _node_tc(h_lig, hacc[0, :n_lig], hacc[1, :n_lig],
                                xacc0, xacc1,
                                xl_pad, z_lig, params['node'], 2000)
    return new_h, new_x_pad[:, :xdim]


# speedup vs baseline: 3.6935x; 1.0584x over previous
import dataclasses
import functools

import jax
import jax.numpy as jnp
from jax import lax
from jax.experimental import pallas as pl
from jax.experimental.pallas import tpu as pltpu
from jax.experimental.pallas import tpu_sc as plsc

NC, NS = 2, 16
NW = NC * NS
SCB = 80
XW = 16
L = 16
NCHUNK = SCB // L


def _iota():
    return lax.iota(jnp.int32, L)


def _sc_params():
    cp = pltpu.CompilerParams()
    if "needs_layout_passes" in pltpu.CompilerParams.__dataclass_fields__:
        cp = dataclasses.replace(cp, needs_layout_passes=False)
    return cp


def _sc_gather(hs_t, hd_t, xs4_t, xd4_t, s_idx, d_idx):
    E = s_idx.shape[0]
    D = hs_t.shape[1]
    ew = E // NW
    nblk = ew // SCB
    main = nblk - (nblk % 2)
    mesh = plsc.VectorSubcoreMesh(core_axis_name="c", subcore_axis_name="s")
    f32 = jnp.float32

    @functools.partial(
        pl.kernel,
        out_type=(
            jax.ShapeDtypeStruct((E, D), f32),
            jax.ShapeDtypeStruct((E, D), f32),
            jax.ShapeDtypeStruct((E * XW,), f32),
            jax.ShapeDtypeStruct((E * XW,), f32),
        ),
        mesh=mesh,
        scratch_types=[
            pltpu.VMEM((2 * SCB,), jnp.int32),
            pltpu.VMEM((2 * SCB,), jnp.int32),
            pltpu.VMEM((SCB, D), f32),
            pltpu.VMEM((SCB, D), f32),
            pltpu.VMEM((SCB, D), f32),
            pltpu.VMEM((SCB, D), f32),
            pltpu.VMEM((SCB * XW,), f32),
            pltpu.VMEM((SCB * XW,), f32),
            pltpu.VMEM((SCB * XW,), f32),
            pltpu.VMEM((SCB * XW,), f32),
            pltpu.VMEM((xs4_t.shape[0],), f32),
            pltpu.VMEM((xd4_t.shape[0],), f32),
            pltpu.SemaphoreType.DMA,
            pltpu.SemaphoreType.DMA,
            pltpu.SemaphoreType.DMA,
        ],
        compiler_params=_sc_params(),
    )
    def k(hs_hbm, hd_hbm, xs4_hbm, xd4_hbm, si_hbm, di_hbm,
          hs_o, hd_o, xsf_o, xdf_o,
          si_v, di_v, hs_b0, hd_b0, hs_b1, hd_b1, xs_b0, xd_b0, xs_b1, xd_b1,
          xs_tile, xd_tile, sem_g0, sem_g1, sem_w):
        wid = lax.axis_index("s") * NC + lax.axis_index("c")
        base0 = wid * ew
        pltpu.sync_copy(xs4_hbm, xs_tile)
        pltpu.sync_copy(xd4_hbm, xd_tile)

        hs_b = (hs_b0, hs_b1)
        hd_b = (hd_b0, hd_b1)
        xs_b = (xs_b0, xs_b1)
        xd_b = (xd_b0, xd_b1)
        sem_g = (sem_g0, sem_g1)

        @pl.loop(0, SCB * XW // L)
        def _(i):
            for b in range(2):
                xs_b[b][pl.ds(i * L, L)] = jnp.zeros((L,), f32)
                xd_b[b][pl.ds(i * L, L)] = jnp.zeros((L,), f32)

        iot = _iota()

        def drain_writes(b):
            pltpu.make_async_copy(hs_o.at[pl.ds(0, SCB)], hs_b[b], sem_w).wait()
            pltpu.make_async_copy(hd_o.at[pl.ds(0, SCB)], hd_b[b], sem_w).wait()
            pltpu.make_async_copy(xsf_o.at[pl.ds(0, SCB * XW)], xs_b[b],
                                  sem_w).wait()
            pltpu.make_async_copy(xdf_o.at[pl.ds(0, SCB * XW)], xd_b[b],
                                  sem_w).wait()

        def x_work(b, col0):
            for kk in range(NCHUNK):
                sc = si_v[pl.ds(col0 + kk * L, L)] << 2
                dc = di_v[pl.ds(col0 + kk * L, L)] << 2
                pos = (iot << 4) + (kk * L * XW)
                for c in range(3):
                    vs = plsc.load_gather(xs_tile, [sc + c])
                    plsc.store_scatter(xs_b[b], [pos + c], vs)
                    vd = plsc.load_gather(xd_tile, [dc + c])
                    plsc.store_scatter(xd_b[b], [pos + c], vd)

        def do_pair(t, drain):
            base = base0 + t * SCB
            if drain:
                for b in range(2):
                    drain_writes(b)
            pltpu.sync_copy(si_hbm.at[pl.ds(base, 2 * SCB)], si_v)
            pltpu.sync_copy(di_hbm.at[pl.ds(base, 2 * SCB)], di_v)
            cps = []
            for b in range(2):
                cps.append(pltpu.async_copy(
                    hs_hbm.at[si_v.at[pl.ds(b * SCB, SCB)]],
                    hs_b[b], sem_g[b]))
                cps.append(pltpu.async_copy(
                    hd_hbm.at[di_v.at[pl.ds(b * SCB, SCB)]],
                    hd_b[b], sem_g[b]))
            for b in range(2):
                x_work(b, b * SCB)
            for b in range(2):
                cps[2 * b].wait()
                cps[2 * b + 1].wait()
                bb = base + b * SCB
                pltpu.async_copy(hs_b[b], hs_o.at[pl.ds(bb, SCB)], sem_w)
                pltpu.async_copy(hd_b[b], hd_o.at[pl.ds(bb, SCB)], sem_w)
                pltpu.async_copy(xs_b[b], xsf_o.at[pl.ds(bb * XW, SCB * XW)],
                                 sem_w)
                pltpu.async_copy(xd_b[b], xdf_o.at[pl.ds(bb * XW, SCB * XW)],
                                 sem_w)

        @pl.loop(0, main, step=2)
        def _(t):
            @pl.when(t > 0)
            def _():
                for b in range(2):
                    drain_writes(b)

            do_pair(t, False)

        for b in range(2):
            drain_writes(b)

        if main < nblk:
            base = base0 + main * SCB
            pltpu.sync_copy(si_hbm.at[pl.ds(base, SCB)],
                            si_v.at[pl.ds(0, SCB)])
            pltpu.sync_copy(di_hbm.at[pl.ds(base, SCB)],
                            di_v.at[pl.ds(0, SCB)])
            c1 = pltpu.async_copy(
                hs_hbm.at[si_v.at[pl.ds(0, SCB)]], hs_b[0], sem_g[0])
            c2 = pltpu.async_copy(
                hd_hbm.at[di_v.at[pl.ds(0, SCB)]], hd_b[0], sem_g[0])
            x_work(0, 0)
            c1.wait()
            c2.wait()
            pltpu.sync_copy(hs_b[0], hs_o.at[pl.ds(base, SCB)])
            pltpu.sync_copy(hd_b[0], hd_o.at[pl.ds(base, SCB)])
            pltpu.sync_copy(xs_b[0], xsf_o.at[pl.ds(base * XW, SCB * XW)])
            pltpu.sync_copy(xd_b[0], xdf_o.at[pl.ds(base * XW, SCB * XW)])

    return k(hs_t, hd_t, xs4_t, xd4_t, s_idx, d_idx)


def _edge_tc(hs, hd, xs, xd, w, bt):
    E, D = hs.shape
    f32 = jnp.float32

    def body(hs_r, hd_r, xs_r, xd_r,
             w1a, w1b, w1c, b1, w2, b2, wa, ba,
             wc1a, wc1b, wc1c, bc1, wc2, bc2, wc3,
             mh_o, mx_o):
        diff = xs_r[...] - xd_r[...]
        dij = jnp.sqrt(jnp.sum(diff * diff, axis=1, keepdims=True))
        hsv, hdv = hs_r[...], hd_r[...]
        bf = jnp.bfloat16
        dot = lambda a, b: jnp.dot(a.astype(bf), b.astype(bf),
                                   preferred_element_type=f32)
        u = dot(hsv, w1a[...]) + dot(hdv, w1b[...]) + dij * w1c[...] + b1[...]
        m = jax.nn.silu(u)
        m2 = jax.nn.silu(dot(m, w2[...]) + b2[...])
        g = jax.nn.sigmoid(jnp.sum(m2 * wa[...], axis=1, keepdims=True)
                           + ba[0, 0])
        mh_o[...] = m2 * g
        v = dot(hsv, wc1a[...]) + dot(hdv, wc1b[...]) + dij * wc1c[...] + bc1[...]
        c = jax.nn.silu(v)
        c2 = jax.nn.silu(dot(c, wc2[...]) + bc2[...])
        cc = jnp.sum(c2 * wc3[...], axis=1, keepdims=True)
        mx_o[...] = cc * diff / (dij + 1.0)

    row = lambda i: (i, 0)
    full = lambda i: (0, 0)
    eb = lambda width: pl.BlockSpec((bt, width), row)
    wspec = lambda a: pl.BlockSpec(a.shape, full)

    w1 = w['W1']
    wc1 = w['Wc1']
    args = (w1[:D], w1[D:2 * D], w1[2 * D:2 * D + 1], w['b1'].reshape(1, D),
            w['W2'], w['b2'].reshape(1, D),
            w['Wa'].reshape(1, D), w['ba'].reshape(1, 1),
            wc1[:D], wc1[D:2 * D], wc1[2 * D:2 * D + 1], w['bc1'].reshape(1, D),
            w['Wc2'], w['bc2'].reshape(1, D), w['Wc3'].reshape(1, D))

    return pl.pallas_call(
        body,
        grid=(E // bt,),
        in_specs=[eb(D), eb(D), eb(XW), eb(XW)] + [wspec(a) for a in args],
        out_specs=[eb(D), eb(XW)],
        out_shape=(jax.ShapeDtypeStruct((E, D), f32),
                   jax.ShapeDtypeStruct((E, XW), f32)),
    )(hs, hd, xs, xd, *args)


def _sc_scatter(mh_ll, mxf_ll, d_ll, mh_kl, mxf_kl, d_kl, n_lig):
    D = mh_ll.shape[1]
    f32 = jnp.float32
    mesh = plsc.VectorSubcoreMesh(core_axis_name="c", subcore_axis_name="s")
    nx = n_lig // 8
    nh_p = -(-n_lig // 128) * 128
    nx_p = -(-nx // 128) * 128
    hr = -(-nh_p // NS) // 8 * 8
    h_parts = [(s * hr, min(hr, nh_p - s * hr)) for s in range(NS)
               if s * hr < nh_p]
    xr = max(8, -(-nx_p // NS) // 8 * 8)
    x_parts = [(s * xr, min(xr, nx_p - s * xr)) for s in range(NS)
               if s * xr < nx_p]
    zeros_h = jnp.zeros((nh_p, D), f32)
    zeros_x = jnp.zeros((nx_p, D), f32)

    @functools.partial(
        pl.kernel,
        out_type=(jax.ShapeDtypeStruct((NC, nh_p, D), f32),
                  jax.ShapeDtypeStruct((NC, nx_p, D), f32)),
        mesh=mesh,
        scratch_types=[
            pltpu.VMEM((SCB,), jnp.int32),
            pltpu.VMEM((SCB,), jnp.int32),
            pltpu.VMEM((SCB, D), f32),
            pltpu.VMEM((SCB * XW,), f32),
            pltpu.VMEM((SCB, D), f32),
            pltpu.VMEM_SHARED((nh_p, D), f32),
            pltpu.VMEM_SHARED((nx_p, D), f32),
        ],
        compiler_params=_sc_params(),
    )
    def k(mhll_hbm, mxll_hbm, dll_hbm, mhkl_hbm, mxkl_hbm, dkl_hbm,
          zh_hbm, zx_hbm, hacc_o, xacc_o,
          di_v, dr_v, mh_v, mx_v, xexp, acc_h, acc_x):
        cid = lax.axis_index("c")
        sid = lax.axis_index("s")
        for s, (off, cnt) in enumerate(h_parts):
            @pl.when(sid == s)
            def _(off=off, cnt=cnt):
                pltpu.sync_copy(zh_hbm.at[pl.ds(off, cnt)],
                                acc_h.at[pl.ds(off, cnt)])
        for s, (off, cnt) in enumerate(x_parts):
            @pl.when(sid == s)
            def _(off=off, cnt=cnt):
                pltpu.sync_copy(zx_hbm.at[pl.ds(off, cnt)],
                                acc_x.at[pl.ds(off, cnt)])

        @pl.loop(0, SCB)
        def _(i):
            for kk in range(D // L):
                xexp[i, pl.ds(kk * L, L)] = jnp.zeros((L,), f32)

        plsc.subcore_barrier()
        iot = _iota()

        for mh_hbm, mxf_hbm, dd_hbm in ((mhll_hbm, mxll_hbm, dll_hbm),
                                        (mhkl_hbm, mxkl_hbm, dkl_hbm)):
            e = dd_hbm.shape[0]
            ew = e // NW
            nblk = ew // SCB
            base0 = cid * (e // NC) + sid * ew

            @pl.loop(0, nblk)
            def _(j):
                base = base0 + j * SCB
                pltpu.sync_copy(dd_hbm.at[pl.ds(base, SCB)], di_v)
                pltpu.sync_copy(mh_hbm.at[pl.ds(base, SCB)], mh_v)
                pltpu.sync_copy(mxf_hbm.at[pl.ds(base * XW, SCB * XW)], mx_v)
                for kk in range(NCHUNK):
                    dc = di_v[pl.ds(kk * L, L)]
                    dr_v[pl.ds(kk * L, L)] = dc >> 3
                    rowi = iot + kk * L
                    slot = (dc & 7) << 4
                    for c in range(3):
                        val = plsc.load_gather(
                            mx_v, [(iot << 4) + (kk * L * XW + c)])
                        plsc.store_scatter(xexp, [rowi, slot + c], val)
                pltpu.sync_copy(mh_v, acc_h.at[di_v], add=True)
                pltpu.sync_copy(xexp, acc_x.at[dr_v], add=True)
                for kk in range(NCHUNK):
                    dc = di_v[pl.ds(kk * L, L)]
                    rowi = iot + kk * L
                    slot = (dc & 7) << 4
                    zv = jnp.zeros((L,), f32)
                    for c in range(3):
                        plsc.store_scatter(xexp, [rowi, slot + c], zv)

        plsc.subcore_barrier()
        for s, (off, cnt) in enumerate(h_parts):
            @pl.when(sid == s)
            def _(off=off, cnt=cnt):
                pltpu.sync_copy(acc_h.at[pl.ds(off, cnt)],
                                hacc_o.at[cid, pl.ds(off, cnt)])
        for s, (off, cnt) in enumerate(x_parts):
            @pl.when(sid == s)
            def _(off=off, cnt=cnt):
                pltpu.sync_copy(acc_x.at[pl.ds(off, cnt)],
                                xacc_o.at[cid, pl.ds(off, cnt)])

    return k(mh_ll, mxf_ll, d_ll, mh_kl, mxf_kl, d_kl, zeros_h, zeros_x)


def _node_tc(h_lig, hacc0, hacc1, xacc0, xacc1, x_pad, z, pn, br):
    n, D = h_lig.shape
    f32 = jnp.float32

    def body(h_r, ha0, ha1, xa0, xa1, xp_r, z_r, wn1a, wn1b, bn1, wn2, bn2,
             nh_o, nx_o):
        zinv = 1.0 / z_r[...]
        hn = (ha0[...] + ha1[...]) * zinv
        xn = (xa0[...] + xa1[...]) * zinv
        hv = h_r[...]
        dot = functools.partial(jnp.dot, preferred_element_type=f32)
        t = jax.nn.silu(dot(hv, wn1a[...]) + dot(hn, wn1b[...]) + bn1[...])
        nh_o[...] = hv + dot(t, wn2[...]) + bn2[...]
        nx_o[...] = xp_r[...] + xn

    row = lambda i: (i, 0)
    full = lambda i: (0, 0)
    rb = lambda width: pl.BlockSpec((br, width), row)
    wspec = lambda a: pl.BlockSpec(a.shape, full)

    wn1 = pn['Wn1']
    args = (wn1[:D], wn1[D:], pn['bn1'].reshape(1, D), pn['Wn2'],
            pn['bn2'].reshape(1, D))

    return pl.pallas_call(
        body,
        grid=(n // br,),
        in_specs=[rb(D), rb(D), rb(D), rb(XW), rb(XW), rb(XW),
                  pl.BlockSpec((br, 1), row)] + [wspec(a) for a in args],
        out_specs=[rb(D), rb(XW)],
        out_shape=(jax.ShapeDtypeStruct((n, D), f32),
                   jax.ShapeDtypeStruct((n, XW), f32)),
    )(h_lig, hacc0, hacc1, xacc0, xacc1, x_pad, z, *args)


def kernel(h_lig, h_kp, x_lig, x_kp, z_lig, edge_index_ll, edge_index_kl,
           params):
    n_lig, D = h_lig.shape
    f32 = jnp.float32
    xdim = x_lig.shape[1]

    xl_pad = jnp.pad(x_lig.astype(f32), ((0, 0), (0, XW - xdim)))
    xk_pad = jnp.pad(x_kp.astype(f32), ((0, 0), (0, XW - xdim)))
    xl4 = jnp.pad(x_lig.astype(f32), ((0, 0), (0, 4 - xdim))).reshape(-1)
    xk4 = jnp.pad(x_kp.astype(f32), ((0, 0), (0, 4 - xdim))).reshape(-1)

    s_ll = edge_index_ll[0].astype(jnp.int32)
    d_ll = edge_index_ll[1].astype(jnp.int32)
    s_kl = edge_index_kl[0].astype(jnp.int32)
    d_kl = edge_index_kl[1].astype(jnp.int32)
    e_ll = s_ll.shape[0]
    e_kl = s_kl.shape[0]

    hs_ll, hd_ll, xsf_ll, xdf_ll = _sc_gather(h_lig, h_lig, xl4, xl4,
                                              s_ll, d_ll)
    hs_kl, hd_kl, xsf_kl, xdf_kl = _sc_gather(h_kp, h_lig, xk4, xl4,
                                              s_kl, d_kl)

    mh_ll, mx_ll = _edge_tc(hs_ll, hd_ll, xsf_ll.reshape(e_ll, XW),
                            xdf_ll.reshape(e_ll, XW), params['ll'], 6400)
    mh_kl, mx_kl = _edge_tc(hs_kl, hd_kl, xsf_kl.reshape(e_kl, XW),
                            xdf_kl.reshape(e_kl, XW), params['kl'], 6400)

    hacc, xacc = _sc_scatter(mh_ll, mx_ll.reshape(-1), d_ll,
                             mh_kl, mx_kl.reshape(-1), d_kl, n_lig)

    xacc0 = xacc[0, :n_lig // 8].reshape(n_lig, XW)
    xacc1 = xacc[1, :n_lig // 8].reshape(n_lig, XW)
    new_h, new_x_pad = _node_tc(h_lig, hacc[0, :n_lig], hacc[1, :n_lig],
                                xacc0, xacc1,
                                xl_pad, z_lig, params['node'], 2000)
    return new_h, new_x_pad[:, :xdim]


# speedup vs baseline: 4.1775x; 1.1310x over previous
import dataclasses
import functools

import jax
import jax.numpy as jnp
from jax import lax
from jax.experimental import pallas as pl
from jax.experimental.pallas import tpu as pltpu
from jax.experimental.pallas import tpu_sc as plsc

NC, NS = 2, 16
NW = NC * NS
SCB = 80
XW = 16
L = 16
NCHUNK = SCB // L


def _iota():
    return lax.iota(jnp.int32, L)


def _sc_params():
    cp = pltpu.CompilerParams()
    if "needs_layout_passes" in pltpu.CompilerParams.__dataclass_fields__:
        cp = dataclasses.replace(cp, needs_layout_passes=False)
    return cp


def _sc_gather(hs_t, hd_t, xs4_t, xd4_t, s_idx, d_idx):
    E = s_idx.shape[0]
    D = hs_t.shape[1]
    ew = E // NW
    nblk = ew // SCB
    main = nblk - (nblk % 2)
    mesh = plsc.VectorSubcoreMesh(core_axis_name="c", subcore_axis_name="s")
    f32 = jnp.float32

    @functools.partial(
        pl.kernel,
        out_type=(
            jax.ShapeDtypeStruct((E, D), f32),
            jax.ShapeDtypeStruct((E, D), f32),
            jax.ShapeDtypeStruct((E * XW,), f32),
            jax.ShapeDtypeStruct((E * XW,), f32),
        ),
        mesh=mesh,
        scratch_types=[
            pltpu.VMEM((2 * SCB,), jnp.int32),
            pltpu.VMEM((2 * SCB,), jnp.int32),
            pltpu.VMEM((SCB, D), f32),
            pltpu.VMEM((SCB, D), f32),
            pltpu.VMEM((SCB, D), f32),
            pltpu.VMEM((SCB, D), f32),
            pltpu.VMEM((SCB * XW,), f32),
            pltpu.VMEM((SCB * XW,), f32),
            pltpu.VMEM((SCB * XW,), f32),
            pltpu.VMEM((SCB * XW,), f32),
            pltpu.VMEM((xs4_t.shape[0],), f32),
            pltpu.VMEM((xd4_t.shape[0],), f32),
            pltpu.SemaphoreType.DMA,
            pltpu.SemaphoreType.DMA,
            pltpu.SemaphoreType.DMA,
        ],
        compiler_params=_sc_params(),
    )
    def k(hs_hbm, hd_hbm, xs4_hbm, xd4_hbm, si_hbm, di_hbm,
          hs_o, hd_o, xsf_o, xdf_o,
          si_v, di_v, hs_b0, hd_b0, hs_b1, hd_b1, xs_b0, xd_b0, xs_b1, xd_b1,
          xs_tile, xd_tile, sem_g0, sem_g1, sem_w):
        wid = lax.axis_index("s") * NC + lax.axis_index("c")
        base0 = wid * ew
        pltpu.sync_copy(xs4_hbm, xs_tile)
        pltpu.sync_copy(xd4_hbm, xd_tile)

        hs_b = (hs_b0, hs_b1)
        hd_b = (hd_b0, hd_b1)
        xs_b = (xs_b0, xs_b1)
        xd_b = (xd_b0, xd_b1)
        sem_g = (sem_g0, sem_g1)

        @pl.loop(0, SCB * XW // L)
        def _(i):
            for b in range(2):
                xs_b[b][pl.ds(i * L, L)] = jnp.zeros((L,), f32)
                xd_b[b][pl.ds(i * L, L)] = jnp.zeros((L,), f32)

        iot = _iota()

        def drain_writes(b):
            pltpu.make_async_copy(hs_o.at[pl.ds(0, SCB)], hs_b[b], sem_w).wait()
            pltpu.make_async_copy(hd_o.at[pl.ds(0, SCB)], hd_b[b], sem_w).wait()
            pltpu.make_async_copy(xsf_o.at[pl.ds(0, SCB * XW)], xs_b[b],
                                  sem_w).wait()
            pltpu.make_async_copy(xdf_o.at[pl.ds(0, SCB * XW)], xd_b[b],
                                  sem_w).wait()

        def x_work(b, col0):
            for kk in range(NCHUNK):
                sc = si_v[pl.ds(col0 + kk * L, L)] << 2
                dc = di_v[pl.ds(col0 + kk * L, L)] << 2
                pos = (iot << 4) + (kk * L * XW)
                for c in range(3):
                    vs = plsc.load_gather(xs_tile, [sc + c])
                    plsc.store_scatter(xs_b[b], [pos + c], vs)
                    vd = plsc.load_gather(xd_tile, [dc + c])
                    plsc.store_scatter(xd_b[b], [pos + c], vd)

        def do_pair(t, drain):
            base = base0 + t * SCB
            if drain:
                for b in range(2):
                    drain_writes(b)
            pltpu.sync_copy(si_hbm.at[pl.ds(base, 2 * SCB)], si_v)
            pltpu.sync_copy(di_hbm.at[pl.ds(base, 2 * SCB)], di_v)
            cps = []
            for b in range(2):
                cps.append(pltpu.async_copy(
                    hs_hbm.at[si_v.at[pl.ds(b * SCB, SCB)]],
                    hs_b[b], sem_g[b]))
                cps.append(pltpu.async_copy(
                    hd_hbm.at[di_v.at[pl.ds(b * SCB, SCB)]],
                    hd_b[b], sem_g[b]))
            for b in range(2):
                x_work(b, b * SCB)
            for b in range(2):
                cps[2 * b].wait()
                cps[2 * b + 1].wait()
                bb = base + b * SCB
                pltpu.async_copy(hs_b[b], hs_o.at[pl.ds(bb, SCB)], sem_w)
                pltpu.async_copy(hd_b[b], hd_o.at[pl.ds(bb, SCB)], sem_w)
                pltpu.async_copy(xs_b[b], xsf_o.at[pl.ds(bb * XW, SCB * XW)],
                                 sem_w)
                pltpu.async_copy(xd_b[b], xdf_o.at[pl.ds(bb * XW, SCB * XW)],
                                 sem_w)

        @pl.loop(0, main, step=2)
        def _(t):
            @pl.when(t > 0)
            def _():
                for b in range(2):
                    drain_writes(b)

            do_pair(t, False)

        for b in range(2):
            drain_writes(b)

        if main < nblk:
            base = base0 + main * SCB
            pltpu.sync_copy(si_hbm.at[pl.ds(base, SCB)],
                            si_v.at[pl.ds(0, SCB)])
            pltpu.sync_copy(di_hbm.at[pl.ds(base, SCB)],
                            di_v.at[pl.ds(0, SCB)])
            c1 = pltpu.async_copy(
                hs_hbm.at[si_v.at[pl.ds(0, SCB)]], hs_b[0], sem_g[0])
            c2 = pltpu.async_copy(
                hd_hbm.at[di_v.at[pl.ds(0, SCB)]], hd_b[0], sem_g[0])
            x_work(0, 0)
            c1.wait()
            c2.wait()
            pltpu.sync_copy(hs_b[0], hs_o.at[pl.ds(base, SCB)])
            pltpu.sync_copy(hd_b[0], hd_o.at[pl.ds(base, SCB)])
            pltpu.sync_copy(xs_b[0], xsf_o.at[pl.ds(base * XW, SCB * XW)])
            pltpu.sync_copy(xd_b[0], xdf_o.at[pl.ds(base * XW, SCB * XW)])

    return k(hs_t, hd_t, xs4_t, xd4_t, s_idx, d_idx)


def _edge_tc(hs, hd, xs, xd, w, bt):
    E, D = hs.shape
    f32 = jnp.float32

    def body(hs_r, hd_r, xs_r, xd_r,
             w1a, w1b, w1c, b1, w2, b2, wa, ba,
             wc1a, wc1b, wc1c, bc1, wc2, bc2, wc3,
             mh_o, mx_o):
        diff = xs_r[...] - xd_r[...]
        dij = jnp.sqrt(jnp.sum(diff * diff, axis=1, keepdims=True))
        hsv, hdv = hs_r[...], hd_r[...]
        bf = jnp.bfloat16
        dot = lambda a, b: jnp.dot(a.astype(bf), b.astype(bf),
                                   preferred_element_type=f32)
        u = dot(hsv, w1a[...]) + dot(hdv, w1b[...]) + dij * w1c[...] + b1[...]
        m = jax.nn.silu(u)
        m2 = jax.nn.silu(dot(m, w2[...]) + b2[...])
        g = jax.nn.sigmoid(jnp.sum(m2 * wa[...], axis=1, keepdims=True)
                           + ba[0, 0])
        mh_o[...] = m2 * g
        v = dot(hsv, wc1a[...]) + dot(hdv, wc1b[...]) + dij * wc1c[...] + bc1[...]
        c = jax.nn.silu(v)
        c2 = jax.nn.silu(dot(c, wc2[...]) + bc2[...])
        cc = jnp.sum(c2 * wc3[...], axis=1, keepdims=True)
        mx_o[...] = cc * diff / (dij + 1.0)

    row = lambda i: (i, 0)
    full = lambda i: (0, 0)
    eb = lambda width: pl.BlockSpec((bt, width), row)
    wspec = lambda a: pl.BlockSpec(a.shape, full)

    w1 = w['W1']
    wc1 = w['Wc1']
    args = (w1[:D], w1[D:2 * D], w1[2 * D:2 * D + 1], w['b1'].reshape(1, D),
            w['W2'], w['b2'].reshape(1, D),
            w['Wa'].reshape(1, D), w['ba'].reshape(1, 1),
            wc1[:D], wc1[D:2 * D], wc1[2 * D:2 * D + 1], w['bc1'].reshape(1, D),
            w['Wc2'], w['bc2'].reshape(1, D), w['Wc3'].reshape(1, D))

    return pl.pallas_call(
        body,
        grid=(E // bt,),
        in_specs=[eb(D), eb(D), eb(XW), eb(XW)] + [wspec(a) for a in args],
        out_specs=[eb(D), eb(XW)],
        out_shape=(jax.ShapeDtypeStruct((E, D), f32),
                   jax.ShapeDtypeStruct((E, XW), f32)),
    )(hs, hd, xs, xd, *args)


def _sc_scatter(mh_ll, mxf_ll, d_ll, mh_kl, mxf_kl, d_kl, n_lig):
    D = mh_ll.shape[1]
    f32 = jnp.float32
    mesh = plsc.VectorSubcoreMesh(core_axis_name="c", subcore_axis_name="s")
    nx = n_lig // 8
    nh_p = -(-n_lig // 128) * 128
    nx_p = -(-nx // 128) * 128
    hr = -(-nh_p // NS) // 8 * 8
    h_parts = [(s * hr, min(hr, nh_p - s * hr)) for s in range(NS)
               if s * hr < nh_p]
    xr = max(8, -(-nx_p // NS) // 8 * 8)
    x_parts = [(s * xr, min(xr, nx_p - s * xr)) for s in range(NS)
               if s * xr < nx_p]
    zeros_h = jnp.zeros((nh_p, D), f32)
    zeros_x = jnp.zeros((nx_p, D), f32)

    @functools.partial(
        pl.kernel,
        out_type=(jax.ShapeDtypeStruct((NC, nh_p, D), f32),
                  jax.ShapeDtypeStruct((NC, nx_p, D), f32)),
        mesh=mesh,
        scratch_types=[
            pltpu.VMEM((SCB,), jnp.int32),
            pltpu.VMEM((SCB,), jnp.int32),
            pltpu.VMEM((SCB,), jnp.int32),
            pltpu.VMEM((SCB,), jnp.int32),
            pltpu.VMEM((SCB, D), f32),
            pltpu.VMEM((SCB, D), f32),
            pltpu.VMEM((SCB * XW,), f32),
            pltpu.VMEM((SCB * XW,), f32),
            pltpu.VMEM((SCB, D), f32),
            pltpu.VMEM_SHARED((nh_p, D), f32),
            pltpu.VMEM_SHARED((nx_p, D), f32),
            pltpu.SemaphoreType.DMA,
            pltpu.SemaphoreType.DMA,
            pltpu.SemaphoreType.DMA,
        ],
        compiler_params=_sc_params(),
    )
    def k(mhll_hbm, mxll_hbm, dll_hbm, mhkl_hbm, mxkl_hbm, dkl_hbm,
          zh_hbm, zx_hbm, hacc_o, xacc_o,
          di_v0, di_v1, dr_v0, dr_v1, mh_v0, mh_v1, mx_v0, mx_v1,
          xexp1, acc_h, acc_x, sem_l0, sem_l1, sem_a):
        cid = lax.axis_index("c")
        sid = lax.axis_index("s")
        di_v = (di_v0, di_v1)
        dr_v = (dr_v0, dr_v1)
        mh_v = (mh_v0, mh_v1)
        mx_v = (mx_v0, mx_v1)
        xexp = (xexp1, xexp1)
        sem_l = (sem_l0, sem_l1)
        for s, (off, cnt) in enumerate(h_parts):
            @pl.when(sid == s)
            def _(off=off, cnt=cnt):
                pltpu.sync_copy(zh_hbm.at[pl.ds(off, cnt)],
                                acc_h.at[pl.ds(off, cnt)])
        for s, (off, cnt) in enumerate(x_parts):
            @pl.when(sid == s)
            def _(off=off, cnt=cnt):
                pltpu.sync_copy(zx_hbm.at[pl.ds(off, cnt)],
                                acc_x.at[pl.ds(off, cnt)])

        @pl.loop(0, SCB)
        def _(i):
            for kk in range(D // L):
                xexp1[i, pl.ds(kk * L, L)] = jnp.zeros((L,), f32)

        plsc.subcore_barrier()
        iot = _iota()

        def expand(b):
            for kk in range(NCHUNK):
                dc = di_v[b][pl.ds(kk * L, L)]
                dr_v[b][pl.ds(kk * L, L)] = dc >> 3
                rowi = iot + kk * L
                slot = (dc & 7) << 4
                for c in range(3):
                    val = plsc.load_gather(
                        mx_v[b], [(iot << 4) + (kk * L * XW + c)])
                    plsc.store_scatter(xexp[b], [rowi, slot + c], val)

        def rezero(b):
            zv = jnp.zeros((L,), f32)
            for kk in range(NCHUNK):
                dc = di_v[b][pl.ds(kk * L, L)]
                rowi = iot + kk * L
                slot = (dc & 7) << 4
                for c in range(3):
                    plsc.store_scatter(xexp[b], [rowi, slot + c], zv)

        for mh_hbm, mxf_hbm, dd_hbm in ((mhll_hbm, mxll_hbm, dll_hbm),
                                        (mhkl_hbm, mxkl_hbm, dkl_hbm)):
            e = dd_hbm.shape[0]
            ew = e // NW
            nblk = ew // SCB
            main = nblk - (nblk % 2)
            base0 = cid * (e // NC) + sid * ew

            def load_slot(b, base):
                return [
                    pltpu.async_copy(dd_hbm.at[pl.ds(base, SCB)], di_v[b],
                                     sem_l[b]),
                    pltpu.async_copy(mh_hbm.at[pl.ds(base, SCB)], mh_v[b],
                                     sem_l[b]),
                    pltpu.async_copy(
                        mxf_hbm.at[pl.ds(base * XW, SCB * XW)], mx_v[b],
                        sem_l[b]),
                ]

            def drain_adds(b):
                pltpu.make_async_copy(mh_hbm.at[pl.ds(0, SCB)], mh_v[b],
                                      sem_a).wait()

            @pl.loop(0, main, step=2)
            def _(t):
                @pl.when(t > 0)
                def _():
                    for b in range(2):
                        drain_adds(b)
                cps = [load_slot(b, base0 + (t + b) * SCB) for b in range(2)]
                for b in range(2):
                    for cp in cps[b]:
                        cp.wait()
                    expand(b)
                    pltpu.async_copy(mh_v[b], acc_h.at[di_v[b]], sem_a,
                                     add=True)
                    pltpu.sync_copy(xexp[b], acc_x.at[dr_v[b]], add=True)
                    rezero(b)

            if main > 0:
                for b in range(2):
                    drain_adds(b)

            if main < nblk:
                base = base0 + main * SCB
                for cp in load_slot(0, base):
                    cp.wait()
                expand(0)
                pltpu.sync_copy(mh_v[0], acc_h.at[di_v[0]], add=True)
                pltpu.sync_copy(xexp[0], acc_x.at[dr_v[0]], add=True)
                rezero(0)

        plsc.subcore_barrier()
        for s, (off, cnt) in enumerate(h_parts):
            @pl.when(sid == s)
            def _(off=off, cnt=cnt):
                pltpu.sync_copy(acc_h.at[pl.ds(off, cnt)],
                                hacc_o.at[cid, pl.ds(off, cnt)])
        for s, (off, cnt) in enumerate(x_parts):
            @pl.when(sid == s)
            def _(off=off, cnt=cnt):
                pltpu.sync_copy(acc_x.at[pl.ds(off, cnt)],
                                xacc_o.at[cid, pl.ds(off, cnt)])

    return k(mh_ll, mxf_ll, d_ll, mh_kl, mxf_kl, d_kl, zeros_h, zeros_x)


def _node_tc(h_lig, hacc0, hacc1, xacc0, xacc1, x_pad, z, pn, br):
    n, D = h_lig.shape
    f32 = jnp.float32

    def body(h_r, ha0, ha1, xa0, xa1, xp_r, z_r, wn1a, wn1b, bn1, wn2, bn2,
             nh_o, nx_o):
        zinv = 1.0 / z_r[...]
        hn = (ha0[...] + ha1[...]) * zinv
        xn = (xa0[...] + xa1[...]) * zinv
        hv = h_r[...]
        dot = functools.partial(jnp.dot, preferred_element_type=f32)
        t = jax.nn.silu(dot(hv, wn1a[...]) + dot(hn, wn1b[...]) + bn1[...])
        nh_o[...] = hv + dot(t, wn2[...]) + bn2[...]
        nx_o[...] = xp_r[...] + xn

    row = lambda i: (i, 0)
    full = lambda i: (0, 0)
    rb = lambda width: pl.BlockSpec((br, width), row)
    wspec = lambda a: pl.BlockSpec(a.shape, full)

    wn1 = pn['Wn1']
    args = (wn1[:D], wn1[D:], pn['bn1'].reshape(1, D), pn['Wn2'],
            pn['bn2'].reshape(1, D))

    return pl.pallas_call(
        body,
        grid=(n // br,),
        in_specs=[rb(D), rb(D), rb(D), rb(XW), rb(XW), rb(XW),
                  pl.BlockSpec((br, 1), row)] + [wspec(a) for a in args],
        out_specs=[rb(D), rb(XW)],
        out_shape=(jax.ShapeDtypeStruct((n, D), f32),
                   jax.ShapeDtypeStruct((n, XW), f32)),
    )(h_lig, hacc0, hacc1, xacc0, xacc1, x_pad, z, *args)


def kernel(h_lig, h_kp, x_lig, x_kp, z_lig, edge_index_ll, edge_index_kl,
           params):
    n_lig, D = h_lig.shape
    f32 = jnp.float32
    xdim = x_lig.shape[1]

    xl_pad = jnp.pad(x_lig.astype(f32), ((0, 0), (0, XW - xdim)))
    xk_pad = jnp.pad(x_kp.astype(f32), ((0, 0), (0, XW - xdim)))
    xl4 = jnp.pad(x_lig.astype(f32), ((0, 0), (0, 4 - xdim))).reshape(-1)
    xk4 = jnp.pad(x_kp.astype(f32), ((0, 0), (0, 4 - xdim))).reshape(-1)

    s_ll = edge_index_ll[0].astype(jnp.int32)
    d_ll = edge_index_ll[1].astype(jnp.int32)
    s_kl = edge_index_kl[0].astype(jnp.int32)
    d_kl = edge_index_kl[1].astype(jnp.int32)
    e_ll = s_ll.shape[0]
    e_kl = s_kl.shape[0]

    hs_ll, hd_ll, xsf_ll, xdf_ll = _sc_gather(h_lig, h_lig, xl4, xl4,
                                              s_ll, d_ll)
    hs_kl, hd_kl, xsf_kl, xdf_kl = _sc_gather(h_kp, h_lig, xk4, xl4,
                                              s_kl, d_kl)

    mh_ll, mx_ll = _edge_tc(hs_ll, hd_ll, xsf_ll.reshape(e_ll, XW),
                            xdf_ll.reshape(e_ll, XW), params['ll'], 6400)
    mh_kl, mx_kl = _edge_tc(hs_kl, hd_kl, xsf_kl.reshape(e_kl, XW),
                            xdf_kl.reshape(e_kl, XW), params['kl'], 6400)

    hacc, xacc = _sc_scatter(mh_ll, mx_ll.reshape(-1), d_ll,
                             mh_kl, mx_kl.reshape(-1), d_kl, n_lig)

    xacc0 = xacc[0, :n_lig // 8].reshape(n_lig, XW)
    xacc1 = xacc[1, :n_lig // 8].reshape(n_lig, XW)
    new_h, new_x_pad = _node_tc(h_lig, hacc[0, :n_lig], hacc[1, :n_lig],
                                xacc0, xacc1,
                                xl_pad, z_lig, params['node'], 2000)
    return new_h, new_x_pad[:, :xdim]
